# Initial kernel scaffold; baseline (speedup 1.0000x reference)
import jax
import jax.numpy as jnp
from jax.experimental import pallas as pl


def kernel(tgt_img, y_tgt_pr, mask, weak_mask):
    raise NotImplementedError("write your pallas kernel here")


# speedup vs baseline: 3.9236x; 3.9236x over previous
import jax
import jax.numpy as jnp
import numpy as np
from jax.experimental import pallas as pl
from jax.experimental.pallas import tpu as pltpu

NUM_L = 8
R = 2048
C = 128
LOG_N = 18
LOG_C = 7

_SENT = np.int32(0x7FFFFFFF)


def _roll(x, shift, axis):
    return pltpu.roll(x, shift % x.shape[axis], axis)


def _bitonic_sort(keys):
    row = jax.lax.broadcasted_iota(jnp.int32, (R, C), 0)
    col = jax.lax.broadcasted_iota(jnp.int32, (R, C), 1)
    for k in range(1, LOG_N + 1):
        if k < LOG_C:
            asc = ((col >> k) & 1) == 0
        else:
            asc = ((row >> (k - LOG_C)) & 1) == 0
        for j in reversed(range(k)):
            if j < LOG_C:
                d = 1 << j
                low = ((col >> j) & 1) == 0
                pm = _roll(keys, -d, 2)
                pp = _roll(keys, d, 2)
            else:
                dr = 1 << (j - LOG_C)
                low = ((row >> (j - LOG_C)) & 1) == 0
                pm = _roll(keys, -dr, 1)
                pp = _roll(keys, dr, 1)
            partner = jnp.where(low, pm, pp)
            take_max = jnp.logical_xor(low, asc)
            keys = jnp.where(take_max,
                             jnp.maximum(keys, partner),
                             jnp.minimum(keys, partner))
    return keys


def _rp_loss_kernel(y_ref, wm_ref, out_ref):
    b = pl.program_id(0)
    y = y_ref[0]
    wm = wm_ref[0]
    y = jnp.where(y == 0.0, jnp.float32(0.0), y)
    yb = jax.lax.bitcast_convert_type(y, jnp.int32)
    keys = jnp.concatenate(
        [jnp.where(wm == l, yb, _SENT)[None] for l in range(1, NUM_L)], axis=0)
    skeys = _bitonic_sort(keys)
    r1 = _roll(skeys, 1, 2)
    r2 = _roll(r1, 1, 1)
    row = jax.lax.broadcasted_iota(jnp.int32, (R, C), 0)
    col = jax.lax.broadcasted_iota(jnp.int32, (R, C), 1)
    prev = jnp.where(col == 0, r2, r1)
    first = (row == 0) & (col == 0)
    real = skeys != _SENT
    start = jnp.logical_and(real, jnp.logical_or(skeys != prev, first))
    contrib = jnp.float32(0.0)
    num_present = jnp.int32(0)
    for i in range(NUM_L - 1):
        distinct = jnp.sum(start[i].astype(jnp.int32))
        n_l = jnp.sum(real[i].astype(jnp.int32))
        has = n_l > 0
        contrib = contrib + jnp.where(has, distinct - 1, 0).astype(jnp.float32)
        num_present = num_present + has.astype(jnp.int32)
    prev_loss = jnp.where(b == 0, jnp.float32(0.0), out_ref[0, 0])
    loss = prev_loss + contrib
    loss = jnp.where(num_present > 0,
                     loss / num_present.astype(jnp.float32), loss)
    out_ref[0, 0] = loss


@jax.jit
def kernel(tgt_img, y_tgt_pr, mask, weak_mask):
    del tgt_img, mask
    B = y_tgt_pr.shape[0]
    y = y_tgt_pr.reshape(B, R, C)
    wm = weak_mask.reshape(B, R, C)
    out = pl.pallas_call(
        _rp_loss_kernel,
        grid=(B,),
        in_specs=[
            pl.BlockSpec((1, R, C), lambda b: (b, 0, 0)),
            pl.BlockSpec((1, R, C), lambda b: (b, 0, 0)),
        ],
        out_specs=pl.BlockSpec(memory_space=pltpu.SMEM),
        out_shape=jax.ShapeDtypeStruct((1, 1), jnp.float32),
    )(y, wm)
    return out[0, 0]


# speedup vs baseline: 9.0011x; 2.2941x over previous
import jax
import jax.numpy as jnp
import numpy as np
from jax.experimental import pallas as pl
from jax.experimental.pallas import tpu as pltpu

NUM_L = 8
R = 2048
C = 128
LOG_N = 18
LOG_C = 7


def _roll(x, shift, axis):
    return pltpu.roll(x, shift % x.shape[axis], axis)


def _iotas():
    row = jax.lax.broadcasted_iota(jnp.int32, (R, C), 0)
    col = jax.lax.broadcasted_iota(jnp.int32, (R, C), 1)
    return row, col


def _bitonic_sort_pairs(keys, vals):
    row, col = _iotas()
    for k in range(1, LOG_N + 1):
        if k < LOG_C:
            asc = ((col >> k) & 1) == 0
        else:
            asc = ((row >> (k - LOG_C)) & 1) == 0
        for j in reversed(range(k)):
            if j < LOG_C:
                d = 1 << j
                low = ((col >> j) & 1) == 0
                pk = jnp.where(low, _roll(keys, -d, 1), _roll(keys, d, 1))
                pv = jnp.where(low, _roll(vals, -d, 1), _roll(vals, d, 1))
            else:
                dr = 1 << (j - LOG_C)
                low = ((row >> (j - LOG_C)) & 1) == 0
                pk = jnp.where(low, _roll(keys, -dr, 0), _roll(keys, dr, 0))
                pv = jnp.where(low, _roll(vals, -dr, 0), _roll(vals, dr, 0))
            take_max = jnp.logical_xor(low, asc)
            take_partner = jnp.logical_or(
                jnp.logical_and(take_max, pk > keys),
                jnp.logical_and(jnp.logical_not(take_max), pk < keys))
            keys = jnp.where(take_partner, pk, keys)
            vals = jnp.where(take_partner, pv, vals)
    return keys, vals


def _shift_down(x, s, fill):
    row, col = _iotas()
    if s < C:
        r1 = _roll(x, s, 1)
        r2 = _roll(r1, 1, 0)
        out = jnp.where(col < s, r2, r1)
        return jnp.where(jnp.logical_and(row == 0, col < s), fill, out)
    sr = s // C
    out = _roll(x, sr, 0)
    return jnp.where(row < sr, fill, out)


def _rp_loss_kernel(y_ref, wm_ref, out_ref):
    b = pl.program_id(0)
    y = y_ref[0]
    wm = wm_ref[0]
    y = jnp.where(y == 0.0, jnp.float32(0.0), y)
    yb = jax.lax.bitcast_convert_type(y, jnp.int32)
    skeys, slab = _bitonic_sort_pairs(yb, wm)
    row, col = _iotas()
    first = jnp.logical_and(row == 0, col == 0)
    prev_k = _shift_down(skeys, 1, np.int32(0))
    f = jnp.logical_or(skeys != prev_k, first)
    bm = jnp.zeros((R, C), jnp.int32)
    for l in range(NUM_L):
        bm = jnp.where(slab == l, np.int32(1 << l), bm)
    m = bm
    fc = f.astype(jnp.int32)
    for p in range(LOG_N):
        s = 1 << p
        ms = _shift_down(m, s, np.int32(0))
        fs = _shift_down(fc, s, np.int32(1))
        m = jnp.where(fc != 0, m, m | ms)
        fc = fc | fs
    e = jnp.where(f, np.int32(0), _shift_down(m, 1, np.int32(0)))
    first_occ = (e & bm) == 0
    contrib = jnp.float32(0.0)
    num_present = jnp.int32(0)
    for l in range(1, NUM_L):
        is_l = slab == l
        distinct = jnp.sum(jnp.logical_and(is_l, first_occ).astype(jnp.int32))
        n_l = jnp.sum(is_l.astype(jnp.int32))
        has = n_l > 0
        contrib = contrib + jnp.where(has, distinct - 1, 0).astype(jnp.float32)
        num_present = num_present + has.astype(jnp.int32)
    prev_loss = jnp.where(b == 0, jnp.float32(0.0), out_ref[0, 0])
    loss = prev_loss + contrib
    loss = jnp.where(num_present > 0,
                     loss / num_present.astype(jnp.float32), loss)
    out_ref[0, 0] = loss


@jax.jit
def kernel(tgt_img, y_tgt_pr, mask, weak_mask):
    del tgt_img, mask
    B = y_tgt_pr.shape[0]
    y = y_tgt_pr.reshape(B, R, C)
    wm = weak_mask.reshape(B, R, C)
    out = pl.pallas_call(
        _rp_loss_kernel,
        grid=(B,),
        in_specs=[
            pl.BlockSpec((1, R, C), lambda b: (b, 0, 0)),
            pl.BlockSpec((1, R, C), lambda b: (b, 0, 0)),
        ],
        out_specs=pl.BlockSpec(memory_space=pltpu.SMEM),
        out_shape=jax.ShapeDtypeStruct((1, 1), jnp.float32),
    )(y, wm)
    return out[0, 0]
